# Initial kernel scaffold; baseline (speedup 1.0000x reference)
import jax
import jax.numpy as jnp
from jax.experimental import pallas as pl


def kernel(pc1, pc2, pred_flows):
    raise NotImplementedError("write your pallas kernel here")


# speedup vs baseline: 35.1270x; 35.1270x over previous
import functools

import jax
import jax.numpy as jnp
from jax.experimental import pallas as pl
from jax.experimental.pallas import tpu as pltpu

_NI = 16
_N = 2048
_RB = 256
_NRB = _N // _RB
_INF = float("inf")


def _sqdist(x, yt):
    xn = jnp.sum(x * x, axis=1, keepdims=True)
    yn = jnp.sum(yt * yt, axis=0, keepdims=True)
    prod = jnp.dot(x.astype(jnp.bfloat16), yt.astype(jnp.bfloat16),
                   preferred_element_type=jnp.float32)
    return (-2.0 * prod + xn) + yn


def _kth_smallest(D, k, also=None):
    d = D
    extras = {}
    m = None
    for p in range(1, k + 1):
        m = jnp.min(d, axis=1, keepdims=True)
        if also and p in also:
            extras[p] = m
        if p < k:
            d = jnp.where(d <= m, _INF, d)
    return m, extras


def _masked_rowsum(mask, row_c):
    return jnp.sum(mask * row_c, axis=1, keepdims=True)


def _curv_kernel(x_ref, xt_ref, out_ref):
    x = x_ref[0]
    xt = xt_ref[0]
    D = _sqdist(x, xt)
    t10, _ = _kth_smallest(D, 10)
    mask = (D <= t10).astype(jnp.float32)
    cnt = jnp.sum(mask, axis=1, keepdims=True)
    cols = [(_masked_rowsum(mask, xt[c:c + 1, :]) - cnt * x[:, c:c + 1]) / 9.0
            for c in range(3)]
    out_ref[0] = jnp.concatenate(cols, axis=1)


def _loss_kernel(x1_ref, fl_ref, x1t_ref, flt_ref, x2t_ref, curv2_ref,
                 out_ref, colmin_ref):
    r = pl.program_id(1)
    x1 = x1_ref[0]
    fl = fl_ref[0]
    x1t = x1t_ref[0]
    flt = flt_ref[0]
    x2t = x2t_ref[0]
    w1 = x1 + fl
    w1t = x1t + flt

    D11 = _sqdist(x1, x1t)
    t10, ex = _kth_smallest(D11, 10, also=(9,))
    t9 = ex[9]
    M10 = (D11 <= t10).astype(jnp.float32)
    cnt10 = jnp.sum(M10, axis=1, keepdims=True)
    M9 = (D11 <= t9).astype(jnp.float32)
    moved = jnp.concatenate(
        [(_masked_rowsum(M10, w1t[c:c + 1, :]) - cnt10 * w1[:, c:c + 1]) / 9.0
         for c in range(3)], axis=1)
    nd = ((flt[0:1, :] - fl[:, 0:1]) ** 2 +
          (flt[1:2, :] - fl[:, 1:2]) ** 2 +
          (flt[2:3, :] - fl[:, 2:3]) ** 2)
    flow_norm = jnp.sqrt(jnp.maximum(nd, 1e-24))
    smooth_part = jnp.sum(M9 * flow_norm) / 8.0

    D12 = _sqdist(w1, x2t)
    t5, ex = _kth_smallest(D12, 5, also=(1,))
    dist1_part = jnp.sum(ex[1])
    M5 = (D12 <= t5).astype(jnp.float32)
    wgt = M5 / (D12 + 1e-8)
    norm = jnp.sum(wgt, axis=1, keepdims=True)
    inter = jnp.dot(wgt, curv2_ref[0], preferred_element_type=jnp.float32,
                    precision=jax.lax.Precision.HIGHEST) / norm
    curv_part = jnp.sum((inter - moved) ** 2)

    cm = jnp.min(D12, axis=0, keepdims=True)
    @pl.when(r == 0)
    def _():
        colmin_ref[...] = cm

    @pl.when(r > 0)
    def _():
        colmin_ref[...] = jnp.minimum(colmin_ref[...], cm)

    part = dist1_part + smooth_part + 0.3 * curv_part
    prev = jnp.where(r == 0, jnp.zeros((1, 1, 1), jnp.float32), out_ref[...])
    total = prev + jnp.reshape(part, (1, 1, 1))
    total = jnp.where(r == _NRB - 1,
                      total + jnp.sum(colmin_ref[...]),
                      total)
    out_ref[...] = total


def _tall_spec():
    return pl.BlockSpec((1, _RB, 3), lambda i, r: (i, r, 0))


def _wide_spec():
    return pl.BlockSpec((1, 3, _N), lambda i, r: (i, 0, 0))


@jax.jit
def kernel(pc1, pc2, pred_flows):
    p1 = pc1.reshape(_NI, _N, 3)
    p2 = pc2.reshape(_NI, _N, 3)
    fl = pred_flows.reshape(_NI, _N, 3)
    p1t = jnp.swapaxes(p1, 1, 2)
    p2t = jnp.swapaxes(p2, 1, 2)
    flt = jnp.swapaxes(fl, 1, 2)

    curv2 = pl.pallas_call(
        _curv_kernel,
        grid=(_NI, _NRB),
        in_specs=[_tall_spec(), _wide_spec()],
        out_specs=pl.BlockSpec((1, _RB, 3), lambda i, r: (i, r, 0)),
        out_shape=jax.ShapeDtypeStruct((_NI, _N, 3), jnp.float32),
    )(p2, p2t)

    partials = pl.pallas_call(
        _loss_kernel,
        grid=(_NI, _NRB),
        in_specs=[_tall_spec(), _tall_spec(), _wide_spec(), _wide_spec(),
                  _wide_spec(),
                  pl.BlockSpec((1, _N, 3), lambda i, r: (i, 0, 0))],
        out_specs=pl.BlockSpec((1, 1, 1), lambda i, r: (i, 0, 0)),
        out_shape=jax.ShapeDtypeStruct((_NI, 1, 1), jnp.float32),
        scratch_shapes=[pltpu.VMEM((1, _N), jnp.float32)],
    )(p1, fl, p1t, flt, p2t, curv2)

    weights = jnp.repeat(jnp.array([0.02, 0.04, 0.08, 0.16], jnp.float32), 4) / 4.0
    return jnp.sum(partials[:, 0, 0] * weights)


# speedup vs baseline: 35.5199x; 1.0112x over previous
import functools

import jax
import jax.numpy as jnp
from jax.experimental import pallas as pl
from jax.experimental.pallas import tpu as pltpu

_NI = 16
_N = 2048
_RB = 256
_NRB = _N // _RB
_INF = float("inf")


def _sqdist(x, yt):
    xn = jnp.sum(x * x, axis=1, keepdims=True)
    yn = jnp.sum(yt * yt, axis=0, keepdims=True)
    prod = jnp.dot(x.astype(jnp.bfloat16), yt.astype(jnp.bfloat16),
                   preferred_element_type=jnp.float32)
    return (-2.0 * prod + xn) + yn


def _kth_smallest(D, k, also=None):
    extras = {}
    m = jnp.min(D, axis=1, keepdims=True)
    if also and 1 in also:
        extras[1] = m
    for p in range(2, k + 1):
        m = jnp.min(jnp.where(D > m, D, _INF), axis=1, keepdims=True)
        if also and p in also:
            extras[p] = m
    return m, extras


def _masked_rowsum(mask, row_c):
    return jnp.sum(mask * row_c, axis=1, keepdims=True)


def _curv_kernel(x_ref, xt_ref, out_ref):
    x = x_ref[0]
    xt = xt_ref[0]
    D = _sqdist(x, xt)
    t10, _ = _kth_smallest(D, 10)
    mask = (D <= t10).astype(jnp.float32)
    cnt = jnp.sum(mask, axis=1, keepdims=True)
    cols = [(_masked_rowsum(mask, xt[c:c + 1, :]) - cnt * x[:, c:c + 1]) / 9.0
            for c in range(3)]
    out_ref[0] = jnp.concatenate(cols, axis=1)


def _loss_kernel(x1_ref, fl_ref, x1t_ref, flt_ref, x2t_ref, curv2_ref,
                 out_ref, colmin_ref):
    r = pl.program_id(1)
    x1 = x1_ref[0]
    fl = fl_ref[0]
    x1t = x1t_ref[0]
    flt = flt_ref[0]
    x2t = x2t_ref[0]
    w1 = x1 + fl
    w1t = x1t + flt

    D11 = _sqdist(x1, x1t)
    t10, ex = _kth_smallest(D11, 10, also=(9,))
    t9 = ex[9]
    M10 = (D11 <= t10).astype(jnp.float32)
    cnt10 = jnp.sum(M10, axis=1, keepdims=True)
    M9 = (D11 <= t9).astype(jnp.float32)
    moved = jnp.concatenate(
        [(_masked_rowsum(M10, w1t[c:c + 1, :]) - cnt10 * w1[:, c:c + 1]) / 9.0
         for c in range(3)], axis=1)
    nd = ((flt[0:1, :] - fl[:, 0:1]) ** 2 +
          (flt[1:2, :] - fl[:, 1:2]) ** 2 +
          (flt[2:3, :] - fl[:, 2:3]) ** 2)
    flow_norm = jnp.sqrt(jnp.maximum(nd, 1e-24))
    smooth_part = jnp.sum(M9 * flow_norm) / 8.0

    D12 = _sqdist(w1, x2t)
    t5, ex = _kth_smallest(D12, 5, also=(1,))
    dist1_part = jnp.sum(ex[1])
    M5 = (D12 <= t5).astype(jnp.float32)
    wgt = M5 / (D12 + 1e-8)
    norm = jnp.sum(wgt, axis=1, keepdims=True)
    inter = jnp.dot(wgt, curv2_ref[0], preferred_element_type=jnp.float32,
                    precision=jax.lax.Precision.HIGHEST) / norm
    curv_part = jnp.sum((inter - moved) ** 2)

    cm = jnp.min(D12, axis=0, keepdims=True)
    @pl.when(r == 0)
    def _():
        colmin_ref[...] = cm

    @pl.when(r > 0)
    def _():
        colmin_ref[...] = jnp.minimum(colmin_ref[...], cm)

    part = dist1_part + smooth_part + 0.3 * curv_part
    prev = jnp.where(r == 0, jnp.zeros((1, 1, 1), jnp.float32), out_ref[...])
    total = prev + jnp.reshape(part, (1, 1, 1))
    total = jnp.where(r == _NRB - 1,
                      total + jnp.sum(colmin_ref[...]),
                      total)
    out_ref[...] = total


def _tall_spec():
    return pl.BlockSpec((1, _RB, 3), lambda i, r: (i, r, 0))


def _wide_spec():
    return pl.BlockSpec((1, 3, _N), lambda i, r: (i, 0, 0))


@jax.jit
def kernel(pc1, pc2, pred_flows):
    p1 = pc1.reshape(_NI, _N, 3)
    p2 = pc2.reshape(_NI, _N, 3)
    fl = pred_flows.reshape(_NI, _N, 3)
    p1t = jnp.swapaxes(p1, 1, 2)
    p2t = jnp.swapaxes(p2, 1, 2)
    flt = jnp.swapaxes(fl, 1, 2)

    curv2 = pl.pallas_call(
        _curv_kernel,
        grid=(_NI, _NRB),
        in_specs=[_tall_spec(), _wide_spec()],
        out_specs=pl.BlockSpec((1, _RB, 3), lambda i, r: (i, r, 0)),
        out_shape=jax.ShapeDtypeStruct((_NI, _N, 3), jnp.float32),
    )(p2, p2t)

    partials = pl.pallas_call(
        _loss_kernel,
        grid=(_NI, _NRB),
        in_specs=[_tall_spec(), _tall_spec(), _wide_spec(), _wide_spec(),
                  _wide_spec(),
                  pl.BlockSpec((1, _N, 3), lambda i, r: (i, 0, 0))],
        out_specs=pl.BlockSpec((1, 1, 1), lambda i, r: (i, 0, 0)),
        out_shape=jax.ShapeDtypeStruct((_NI, 1, 1), jnp.float32),
        scratch_shapes=[pltpu.VMEM((1, _N), jnp.float32)],
    )(p1, fl, p1t, flt, p2t, curv2)

    weights = jnp.repeat(jnp.array([0.02, 0.04, 0.08, 0.16], jnp.float32), 4) / 4.0
    return jnp.sum(partials[:, 0, 0] * weights)


# speedup vs baseline: 40.7649x; 1.1477x over previous
import functools

import jax
import jax.numpy as jnp
from jax.experimental import pallas as pl
from jax.experimental.pallas import tpu as pltpu

_NI = 16
_N = 2048
_RB = 256
_NRB = _N // _RB
_INF = float("inf")


def _sqdist(x, yt):
    xn = jnp.sum(x * x, axis=1, keepdims=True)
    yn = jnp.sum(yt * yt, axis=0, keepdims=True)
    prod = jnp.dot(x.astype(jnp.bfloat16), yt.astype(jnp.bfloat16),
                   preferred_element_type=jnp.float32)
    return (-2.0 * prod + xn) + yn


def _kth_smallest(D, k, also=None):
    extras = {}
    m = jnp.min(D, axis=1, keepdims=True)
    if also and 1 in also:
        extras[1] = m
    for p in range(2, k + 1):
        m = jnp.min(jnp.where(D > m, D, _INF), axis=1, keepdims=True)
        if also and p in also:
            extras[p] = m
    return m, extras


def _masked_rowsum(mask, row_c):
    return jnp.sum(mask * row_c, axis=1, keepdims=True)


def _curv_kernel(x_ref, xt_ref, out_ref):
    x = x_ref[0]
    xt = xt_ref[0]
    D = _sqdist(x, xt)
    t10, _ = _kth_smallest(D, 10)
    mask = (D <= t10).astype(jnp.float32)
    cnt = jnp.sum(mask, axis=1, keepdims=True)
    cols = [(_masked_rowsum(mask, xt[c:c + 1, :]) - cnt * x[:, c:c + 1]) / 9.0
            for c in range(3)]
    out_ref[0] = jnp.concatenate(cols, axis=1)


def _loss_kernel(x1_ref, fl_ref, x1t_ref, flt_ref, x2t_ref, curv2_ref,
                 out_ref, colmin_ref):
    r = pl.program_id(1)
    x1 = x1_ref[0]
    fl = fl_ref[0]
    x1t = x1t_ref[0]
    flt = flt_ref[0]
    x2t = x2t_ref[0]
    w1 = x1 + fl
    w1t = x1t + flt

    D11 = _sqdist(x1, x1t)
    t10, ex = _kth_smallest(D11, 10, also=(9,))
    t9 = ex[9]
    M10 = (D11 <= t10).astype(jnp.float32)
    cnt10 = jnp.sum(M10, axis=1, keepdims=True)
    M9 = (D11 <= t9).astype(jnp.float32)
    movedm = jax.lax.dot_general(
        M10.astype(jnp.bfloat16), w1t.astype(jnp.bfloat16),
        (((1,), (1,)), ((), ())), preferred_element_type=jnp.float32)
    moved = (movedm - cnt10 * w1) / 9.0
    nd = ((flt[0:1, :] - fl[:, 0:1]) ** 2 +
          (flt[1:2, :] - fl[:, 1:2]) ** 2 +
          (flt[2:3, :] - fl[:, 2:3]) ** 2)
    flow_norm = jnp.sqrt(jnp.maximum(nd, 1e-24))
    smooth_part = jnp.sum(M9 * flow_norm) / 8.0

    D12 = _sqdist(w1, x2t)
    t5, ex = _kth_smallest(D12, 5, also=(1,))
    dist1_part = jnp.sum(ex[1])
    M5 = (D12 <= t5).astype(jnp.float32)
    wgt = M5 / (D12 + 1e-8)
    norm = jnp.sum(wgt, axis=1, keepdims=True)
    rnorm = 1.0 / norm
    cv2t = curv2_ref[0]
    curv_part = 0.0
    for c in range(3):
        inter_c = jnp.sum(wgt * cv2t[c:c + 1, :], axis=1, keepdims=True)
        curv_part += jnp.sum((inter_c * rnorm - moved[:, c:c + 1]) ** 2)

    cm = jnp.min(D12, axis=0, keepdims=True)
    @pl.when(r == 0)
    def _():
        colmin_ref[...] = cm

    @pl.when(r > 0)
    def _():
        colmin_ref[...] = jnp.minimum(colmin_ref[...], cm)

    part = dist1_part + smooth_part + 0.3 * curv_part
    prev = jnp.where(r == 0, jnp.zeros((1, 1, 1), jnp.float32), out_ref[...])
    total = prev + jnp.reshape(part, (1, 1, 1))
    total = jnp.where(r == _NRB - 1,
                      total + jnp.sum(colmin_ref[...]),
                      total)
    out_ref[...] = total


def _tall_spec():
    return pl.BlockSpec((1, _RB, 3), lambda i, r: (i, r, 0))


def _wide_spec():
    return pl.BlockSpec((1, 3, _N), lambda i, r: (i, 0, 0))


@jax.jit
def kernel(pc1, pc2, pred_flows):
    p1 = pc1.reshape(_NI, _N, 3)
    p2 = pc2.reshape(_NI, _N, 3)
    fl = pred_flows.reshape(_NI, _N, 3)
    p1t = jnp.swapaxes(p1, 1, 2)
    p2t = jnp.swapaxes(p2, 1, 2)
    flt = jnp.swapaxes(fl, 1, 2)

    curv2 = pl.pallas_call(
        _curv_kernel,
        grid=(_NI, _NRB),
        in_specs=[_tall_spec(), _wide_spec()],
        out_specs=pl.BlockSpec((1, _RB, 3), lambda i, r: (i, r, 0)),
        out_shape=jax.ShapeDtypeStruct((_NI, _N, 3), jnp.float32),
    )(p2, p2t)

    partials = pl.pallas_call(
        _loss_kernel,
        grid=(_NI, _NRB),
        in_specs=[_tall_spec(), _tall_spec(), _wide_spec(), _wide_spec(),
                  _wide_spec(), _wide_spec()],
        out_specs=pl.BlockSpec((1, 1, 1), lambda i, r: (i, 0, 0)),
        out_shape=jax.ShapeDtypeStruct((_NI, 1, 1), jnp.float32),
        scratch_shapes=[pltpu.VMEM((1, _N), jnp.float32)],
    )(p1, fl, p1t, flt, p2t, jnp.swapaxes(curv2, 1, 2))

    weights = jnp.repeat(jnp.array([0.02, 0.04, 0.08, 0.16], jnp.float32), 4) / 4.0
    return jnp.sum(partials[:, 0, 0] * weights)
